# Initial kernel scaffold; baseline (speedup 1.0000x reference)
import jax
import jax.numpy as jnp
from jax.experimental import pallas as pl


def kernel(features, feature_values, emb_table, bias_table, global_bias, W1, b1, W2, b2, Wp):
    raise NotImplementedError("write your pallas kernel here")


# speedup vs baseline: 1.3398x; 1.3398x over previous
import functools

import jax
import jax.numpy as jnp
from jax import lax
from jax.experimental import pallas as pl
from jax.experimental.pallas import tpu as pltpu
from jax.experimental.pallas import tpu_sc as plsc

B = 16384
F = 26
K = 16
NC = 2
NS = 16
NW = NC * NS
IDX_W = 128

TOT = B * F
IDX_ROWS = TOT // IDX_W
ROWS_PER_W = IDX_ROWS // NW
CH = 13
NCHUNK = ROWS_PER_W // CH
BCH = CH * IDX_W // F
GCH = CH * IDX_W


def _fm_body(feat_hbm, fv_hbm, emb_hbm, out_hbm, idx_v, fv_v, rows_v, out_v,
             sem0, sem1, sem_out):
    wid = lax.axis_index("s") * NC + lax.axis_index("c")
    sems = (sem0, sem1)

    def load_and_fire(ch, buf):
        row0 = wid * ROWS_PER_W + ch * CH
        pltpu.sync_copy(feat_hbm.at[pl.ds(row0 * IDX_W, GCH)], idx_v.at[buf])
        pltpu.sync_copy(fv_hbm.at[pl.ds(row0 * IDX_W, GCH)],
                        fv_v.at[buf, pl.ds(0, GCH)])
        return [
            pltpu.async_copy(
                emb_hbm.at[idx_v.at[buf, pl.ds(j * IDX_W, IDX_W)]],
                rows_v.at[buf, pl.ds(j * IDX_W, IDX_W)],
                sems[buf],
            )
            for j in range(CH)
        ]

    out_copies = []
    copies = load_and_fire(0, 0)
    for ch in range(NCHUNK):
        buf = ch % 2
        nxt = load_and_fire(ch + 1, 1 - buf) if ch + 1 < NCHUNK else []
        for c in copies:
            c.wait()

        def body(b, carry):
            base = b * F
            wv0 = fv_v[buf, pl.ds(base, K)]
            wv1 = fv_v[buf, pl.ds(base + K, K)]
            acc = jnp.zeros((K,), jnp.float32)
            acc2 = jnp.zeros((K,), jnp.float32)
            for f in range(F):
                w = wv0[f] if f < K else wv1[f - K]
                wr = rows_v[buf, base + f] * w
                acc = acc + wr
                acc2 = acc2 + wr * wr
            out_v[ch, b] = 0.5 * (acc * acc - acc2)
            return carry

        lax.fori_loop(0, BCH, body, 0)
        b0 = wid * (NCHUNK * BCH) + ch * BCH
        out_copies.append(
            pltpu.async_copy(out_v.at[ch], out_hbm.at[pl.ds(b0, BCH)], sem_out)
        )
        copies = nxt
    for c in out_copies:
        c.wait()


def _fm_sc(feat_flat, fv_flat, emb_table):
    mesh = plsc.VectorSubcoreMesh(core_axis_name="c", subcore_axis_name="s")
    kern = functools.partial(
        pl.kernel,
        out_type=jax.ShapeDtypeStruct((B, K), jnp.float32),
        mesh=mesh,
        scratch_types=[
            pltpu.VMEM((2, GCH), jnp.int32),
            pltpu.VMEM((2, GCH + 2 * K), jnp.float32),
            pltpu.VMEM((2, GCH, K), jnp.float32),
            pltpu.VMEM((NCHUNK, BCH, K), jnp.float32),
            pltpu.SemaphoreType.DMA,
            pltpu.SemaphoreType.DMA,
            pltpu.SemaphoreType.DMA,
        ],
        compiler_params=pltpu.CompilerParams(use_tc_tiling_on_sc=False),
    )(_fm_body)
    return kern(feat_flat, fv_flat, emb_table)


def _mlp_body(fm_ref, w1_ref, b1_ref, w2_ref, b2_ref, wp_ref, gb_ref, out_ref):
    h = jnp.maximum(jnp.dot(fm_ref[...], w1_ref[...],
                            preferred_element_type=jnp.float32) + b1_ref[...], 0.0)
    h = jnp.maximum(jnp.dot(h, w2_ref[...],
                            preferred_element_type=jnp.float32) + b2_ref[...], 0.0)
    p = jnp.dot(h, wp_ref[...], preferred_element_type=jnp.float32)
    out_ref[...] = p + gb_ref[0, 0]


def _mlp_tc(fm, W1, b1, W2, b2, Wp, gb):
    return pl.pallas_call(
        _mlp_body,
        out_shape=jax.ShapeDtypeStruct((B, 1), jnp.float32),
        grid=(4,),
        in_specs=[
            pl.BlockSpec((B // 4, K), lambda i: (i, 0)),
            pl.BlockSpec((K, 64), lambda i: (0, 0)),
            pl.BlockSpec((1, 64), lambda i: (0, 0)),
            pl.BlockSpec((64, 32), lambda i: (0, 0)),
            pl.BlockSpec((1, 32), lambda i: (0, 0)),
            pl.BlockSpec((32, 1), lambda i: (0, 0)),
            pl.BlockSpec((1, 1), lambda i: (0, 0)),
        ],
        out_specs=pl.BlockSpec((B // 4, 1), lambda i: (i, 0)),
    )(fm, W1, b1.reshape(1, -1), W2, b2.reshape(1, -1), Wp, gb.reshape(1, 1))


def kernel(features, feature_values, emb_table, bias_table, global_bias,
           W1, b1, W2, b2, Wp):
    feat_flat = features.astype(jnp.int32).reshape(TOT)
    fv_flat = feature_values.reshape(TOT)
    fm = _fm_sc(feat_flat, fv_flat, emb_table)
    return _mlp_tc(fm, W1, b1, W2, b2, Wp, global_bias).reshape(-1)
